# Initial kernel scaffold; baseline (speedup 1.0000x reference)
import jax
import jax.numpy as jnp
from jax.experimental import pallas as pl


def kernel(input, table):
    raise NotImplementedError("write your pallas kernel here")


# speedup vs baseline: 1.7982x; 1.7982x over previous
import functools

import jax
import jax.numpy as jnp
from jax import lax
from jax.experimental import pallas as pl
from jax.experimental.pallas import tpu as pltpu
from jax.experimental.pallas import tpu_sc as plsc

_VOCAB = 1000000
_DIM = 64
_B = 16384
_L = 50
_N = _B * _L

_NUM_CORES = 2
_NUM_SUBCORES = 16
_NW = _NUM_CORES * _NUM_SUBCORES
_PER_W = _N // _NW
_CHUNK = 512
_NCHUNK = _PER_W // _CHUNK


def _make_gather_kernel():
  mesh = plsc.VectorSubcoreMesh(core_axis_name="c", subcore_axis_name="s")

  @functools.partial(
      pl.kernel,
      mesh=mesh,
      out_type=jax.ShapeDtypeStruct((_N, _DIM), jnp.float32),
      scratch_types=[
          pltpu.VMEM((_CHUNK,), jnp.int32),
          pltpu.VMEM((_CHUNK, _DIM), jnp.float32),
          pltpu.SemaphoreType.DMA,
      ],
      compiler_params=pltpu.CompilerParams(use_tc_tiling_on_sc=False),
  )
  def gather_kernel(idx_hbm, table_hbm, out_hbm, idx_v, rows_v, sem):
    wid = lax.axis_index("s") * _NUM_CORES + lax.axis_index("c")
    base = wid * _PER_W

    def body(i, carry):
      off = base + i * _CHUNK
      pltpu.sync_copy(idx_hbm.at[pl.ds(off, _CHUNK)], idx_v)
      pltpu.async_copy(table_hbm.at[idx_v], rows_v, sem).wait()
      pltpu.sync_copy(rows_v, out_hbm.at[pl.ds(off, _CHUNK)])
      return carry

    lax.fori_loop(0, _NCHUNK, body, 0)

  return gather_kernel


_gather = _make_gather_kernel()


@jax.jit
def kernel(input, table):
  idx = input.reshape(-1).astype(jnp.int32)
  out = _gather(idx, table)
  return out.reshape(_B, _L, _DIM)


# speedup vs baseline: 1.8728x; 1.0415x over previous
import functools

import jax
import jax.numpy as jnp
from jax import lax
from jax.experimental import pallas as pl
from jax.experimental.pallas import tpu as pltpu
from jax.experimental.pallas import tpu_sc as plsc

_VOCAB = 1000000
_DIM = 64
_B = 16384
_L = 50
_N = _B * _L

_NUM_CORES = 2
_NUM_SUBCORES = 16
_NW = _NUM_CORES * _NUM_SUBCORES
_PER_W = _N // _NW
_CHUNK = 800
_NCHUNK = _PER_W // _CHUNK


def _make_gather_kernel():
  mesh = plsc.VectorSubcoreMesh(core_axis_name="c", subcore_axis_name="s")

  @functools.partial(
      pl.kernel,
      mesh=mesh,
      out_type=jax.ShapeDtypeStruct((_N, _DIM), jnp.float32),
      scratch_types=[
          pltpu.VMEM((_PER_W,), jnp.int32),
          pltpu.VMEM((_CHUNK, _DIM), jnp.float32),
          pltpu.VMEM((_CHUNK, _DIM), jnp.float32),
          pltpu.SemaphoreType.DMA,
          pltpu.SemaphoreType.DMA,
          pltpu.SemaphoreType.DMA,
          pltpu.SemaphoreType.DMA,
      ],
      compiler_params=pltpu.CompilerParams(use_tc_tiling_on_sc=False),
  )
  def gather_kernel(idx_hbm, table_hbm, out_hbm, idx_v, rows0, rows1,
                    sem_g0, sem_g1, sem_o0, sem_o1):
    wid = lax.axis_index("s") * _NUM_CORES + lax.axis_index("c")
    base = wid * _PER_W
    rows = (rows0, rows1)
    sem_g = (sem_g0, sem_g1)
    sem_o = (sem_o0, sem_o1)

    def idx_slice(i):
      return idx_v.at[pl.ds(i * _CHUNK, _CHUNK)]

    def out_slice(i):
      return out_hbm.at[pl.ds(base + i * _CHUNK, _CHUNK)]

    def start_gather(i, b):
      pltpu.async_copy(table_hbm.at[idx_slice(i)], rows[b], sem_g[b])

    def wait_gather(i, b):
      pltpu.make_async_copy(table_hbm.at[idx_slice(i)], rows[b],
                            sem_g[b]).wait()

    def start_out(i, b):
      pltpu.async_copy(rows[b], out_slice(i), sem_o[b])

    def wait_out(i, b):
      pltpu.make_async_copy(rows[b], out_slice(i), sem_o[b]).wait()

    pltpu.sync_copy(idx_hbm.at[pl.ds(base, _PER_W)], idx_v)

    start_gather(0, 0)
    start_gather(1, 1)
    wait_gather(0, 0)
    start_out(0, 0)

    def pair_body(g, carry):
      i = 2 * g + 1
      wait_out(i - 1, 0)
      start_gather(i + 1, 0)
      wait_gather(i, 1)
      start_out(i, 1)
      wait_out(i, 1)
      start_gather(i + 2, 1)
      wait_gather(i + 1, 0)
      start_out(i + 1, 0)
      return carry

    lax.fori_loop(0, (_NCHUNK - 2) // 2 - 1, pair_body, 0)

    i = _NCHUNK - 3
    wait_out(i - 1, 0)
    start_gather(i + 1, 0)
    wait_gather(i, 1)
    start_out(i, 1)
    wait_out(i, 1)
    start_gather(i + 2, 1)
    wait_gather(i + 1, 0)
    start_out(i + 1, 0)
    wait_gather(i + 2, 1)
    start_out(i + 2, 1)
    wait_out(i + 1, 0)
    wait_out(i + 2, 1)

  return gather_kernel


_gather = _make_gather_kernel()


@jax.jit
def kernel(input, table):
  idx = input.reshape(-1).astype(jnp.int32)
  out = _gather(idx, table)
  return out.reshape(_B, _L, _DIM)


# speedup vs baseline: 1.8857x; 1.0069x over previous
import functools

import jax
import jax.numpy as jnp
from jax import lax
from jax.experimental import pallas as pl
from jax.experimental.pallas import tpu as pltpu
from jax.experimental.pallas import tpu_sc as plsc

_VOCAB = 1000000
_DIM = 64
_B = 16384
_L = 50
_N = _B * _L

_NUM_CORES = 2
_NUM_SUBCORES = 16
_NW = _NUM_CORES * _NUM_SUBCORES
_PER_W = _N // _NW
_CHUNK = 800
_NCHUNK = _PER_W // _CHUNK


def _make_gather_kernel():
  mesh = plsc.VectorSubcoreMesh(core_axis_name="c", subcore_axis_name="s")

  @functools.partial(
      pl.kernel,
      mesh=mesh,
      out_type=jax.ShapeDtypeStruct((_N, _DIM), jnp.float32),
      scratch_types=[
          pltpu.VMEM((_PER_W,), jnp.int32),
          pltpu.VMEM((_CHUNK, _DIM), jnp.float32),
          pltpu.VMEM((_CHUNK, _DIM), jnp.float32),
          pltpu.SemaphoreType.DMA,
          pltpu.SemaphoreType.DMA,
          pltpu.SemaphoreType.DMA,
          pltpu.SemaphoreType.DMA,
      ],
      compiler_params=pltpu.CompilerParams(use_tc_tiling_on_sc=False),
  )
  def gather_kernel(idx_hbm, table_hbm, out_hbm, idx_v, rows0, rows1,
                    sem_g0, sem_g1, sem_o0, sem_o1):
    wid = lax.axis_index("s") * _NUM_CORES + lax.axis_index("c")
    base = wid * _PER_W
    rows = (rows0, rows1)
    sem_g = (sem_g0, sem_g1)
    sem_o = (sem_o0, sem_o1)

    def idx_slice(i):
      return idx_v.at[pl.ds(i * _CHUNK, _CHUNK)]

    def out_slice(i):
      return out_hbm.at[pl.ds(base + i * _CHUNK, _CHUNK)]

    _NSUB = 4
    _SUB = _CHUNK // _NSUB

    def start_gather(i, b):
      for s in range(_NSUB):
        pltpu.async_copy(
            table_hbm.at[idx_v.at[pl.ds(i * _CHUNK + s * _SUB, _SUB)]],
            rows[b].at[pl.ds(s * _SUB, _SUB)], sem_g[b])

    def wait_gather(i, b):
      for s in range(_NSUB):
        pltpu.make_async_copy(
            table_hbm.at[idx_v.at[pl.ds(i * _CHUNK + s * _SUB, _SUB)]],
            rows[b].at[pl.ds(s * _SUB, _SUB)], sem_g[b]).wait()

    def start_out(i, b):
      pltpu.async_copy(rows[b], out_slice(i), sem_o[b])

    def wait_out(i, b):
      pltpu.make_async_copy(rows[b], out_slice(i), sem_o[b]).wait()

    pltpu.sync_copy(idx_hbm.at[pl.ds(base, _PER_W)], idx_v)

    start_gather(0, 0)
    start_gather(1, 1)
    wait_gather(0, 0)
    start_out(0, 0)

    def pair_body(g, carry):
      i = 2 * g + 1
      wait_out(i - 1, 0)
      start_gather(i + 1, 0)
      wait_gather(i, 1)
      start_out(i, 1)
      wait_out(i, 1)
      start_gather(i + 2, 1)
      wait_gather(i + 1, 0)
      start_out(i + 1, 0)
      return carry

    lax.fori_loop(0, (_NCHUNK - 2) // 2 - 1, pair_body, 0)

    i = _NCHUNK - 3
    wait_out(i - 1, 0)
    start_gather(i + 1, 0)
    wait_gather(i, 1)
    start_out(i, 1)
    wait_out(i, 1)
    start_gather(i + 2, 1)
    wait_gather(i + 1, 0)
    start_out(i + 1, 0)
    wait_gather(i + 2, 1)
    start_out(i + 2, 1)
    wait_out(i + 1, 0)
    wait_out(i + 2, 1)

  return gather_kernel


_gather = _make_gather_kernel()


@jax.jit
def kernel(input, table):
  idx = input.reshape(-1).astype(jnp.int32)
  out = _gather(idx, table)
  return out.reshape(_B, _L, _DIM)
